# baseline (speedup 1.0000x reference)
import functools

import jax
import jax.numpy as jnp
from jax.experimental import pallas as pl
from jax.experimental.pallas import tpu as pltpu

_NUM_Q = 8
_K = 8192
_D = 64
_COMMITMENT = 1.0
_R = 256


def _rvq_block(x_ref, cb_ref, qout_ref, idx_ref, loss_ref, res_ref):
    q = pl.program_id(1)

    @pl.when(q == 0)
    def _init():
        res_ref[...] = x_ref[...]

    r = res_ref[...]
    cb = cb_ref[0]

    rn = jnp.sum(r * r, axis=1, keepdims=True)
    cbn = jnp.sum(cb * cb, axis=1)[None, :]
    mm = jax.lax.dot_general(
        r.astype(jnp.bfloat16), cb.astype(jnp.bfloat16),
        (((1,), (1,)), ((), ())),
        preferred_element_type=jnp.float32)
    neg_dist = -(rn - 2.0 * mm + cbn)

    maxv = jnp.max(neg_dist, axis=1, keepdims=True)
    iota = jax.lax.broadcasted_iota(jnp.int32, (_R, _K), 1)
    idx = jnp.min(jnp.where(neg_dist == maxv, iota, _K), axis=1)

    onehot = (iota == idx[:, None]).astype(jnp.float32)
    quant = jax.lax.dot_general(
        onehot, cb, (((1,), (0,)), ((), ())),
        preferred_element_type=jnp.float32,
        precision=jax.lax.Precision.HIGHEST)

    new_res = r - quant
    res_ref[...] = new_res

    idx_ref[...] = idx.reshape(1, 1, 1, _R)
    loss_ref[...] = jnp.sum(new_res * new_res).reshape(1, 1, 1, 1)

    @pl.when(q == 0)
    def _first():
        qout_ref[...] = quant

    @pl.when(q != 0)
    def _acc():
        qout_ref[...] = qout_ref[...] + quant


@functools.partial(jax.jit, static_argnames=())
def kernel(x, codebooks):
    orig_shape = x.shape
    n = orig_shape[0] * orig_shape[1]
    xf = x.reshape(n, _D)
    nblocks = n // _R

    grid = (nblocks, _NUM_Q)
    qout, idx, loss_part = pl.pallas_call(
        _rvq_block,
        grid=grid,
        in_specs=[
            pl.BlockSpec((_R, _D), lambda i, q: (i, 0)),
            pl.BlockSpec((1, _K, _D), lambda i, q: (q, 0, 0)),
        ],
        out_specs=[
            pl.BlockSpec((_R, _D), lambda i, q: (i, 0)),
            pl.BlockSpec((1, 1, 1, _R), lambda i, q: (q, i, 0, 0)),
            pl.BlockSpec((1, 1, 1, 1), lambda i, q: (i, q, 0, 0)),
        ],
        out_shape=[
            jax.ShapeDtypeStruct((n, _D), jnp.float32),
            jax.ShapeDtypeStruct((_NUM_Q, nblocks, 1, _R), jnp.int32),
            jax.ShapeDtypeStruct((nblocks, _NUM_Q, 1, 1), jnp.float32),
        ],
        scratch_shapes=[pltpu.VMEM((_R, _D), jnp.float32)],
        compiler_params=pltpu.CompilerParams(
            dimension_semantics=("arbitrary", "arbitrary")),
    )(xf, codebooks)

    quantized_out = qout.reshape(orig_shape)
    indices = idx.reshape(_NUM_Q, *orig_shape[:-1])
    losses = loss_part.reshape(nblocks, _NUM_Q).sum(axis=0) * (
        _COMMITMENT / float(n * _D))
    return quantized_out, indices, losses


# speedup vs baseline: 1.4576x; 1.4576x over previous
import functools

import jax
import jax.numpy as jnp
from jax.experimental import pallas as pl
from jax.experimental.pallas import tpu as pltpu

_NUM_Q = 8
_K = 8192
_D = 64
_COMMITMENT = 1.0
_R = 256


def _rvq_block(x_ref, cb_ref, cbhi_ref, cbmid_ref, cblo_ref,
               qout_ref, idx_ref, loss_ref, res_ref):
    q = pl.program_id(1)

    @pl.when(q == 0)
    def _init():
        res_ref[...] = x_ref[...]

    r = res_ref[...]
    cb = cb_ref[0]

    rn = jnp.sum(r * r, axis=1, keepdims=True)
    cbn = jnp.sum(cb * cb, axis=1)[None, :]
    mm = jax.lax.dot_general(
        r.astype(jnp.bfloat16), cb.astype(jnp.bfloat16),
        (((1,), (1,)), ((), ())),
        preferred_element_type=jnp.float32)
    neg_dist = -(rn - 2.0 * mm + cbn)

    maxv = jnp.max(neg_dist, axis=1, keepdims=True)
    iota = jax.lax.broadcasted_iota(jnp.int32, (_R, _K), 1)
    idx = jnp.min(jnp.where(neg_dist == maxv, iota, _K), axis=1)

    onehot = (iota == idx[:, None]).astype(jnp.bfloat16)
    dims = (((1,), (0,)), ((), ()))
    quant = (jax.lax.dot_general(onehot, cbhi_ref[0], dims,
                                 preferred_element_type=jnp.float32)
             + jax.lax.dot_general(onehot, cbmid_ref[0], dims,
                                   preferred_element_type=jnp.float32)
             ) + jax.lax.dot_general(onehot, cblo_ref[0], dims,
                                     preferred_element_type=jnp.float32)

    new_res = r - quant
    res_ref[...] = new_res

    idx_ref[...] = idx.reshape(1, 1, 1, _R)
    loss_ref[...] = jnp.sum(new_res * new_res).reshape(1, 1, 1, 1)

    @pl.when(q == 0)
    def _first():
        qout_ref[...] = quant

    @pl.when(q != 0)
    def _acc():
        qout_ref[...] = qout_ref[...] + quant


@functools.partial(jax.jit, static_argnames=())
def kernel(x, codebooks):
    orig_shape = x.shape
    n = orig_shape[0] * orig_shape[1]
    xf = x.reshape(n, _D)
    nblocks = n // _R

    cb_hi16 = codebooks.astype(jnp.bfloat16)
    cb_hi = cb_hi16.astype(jnp.float32)
    cb_mid16 = (codebooks - cb_hi).astype(jnp.bfloat16)
    cb_mid = cb_mid16.astype(jnp.float32)
    cb_lo16 = (codebooks - cb_hi - cb_mid).astype(jnp.bfloat16)

    grid = (nblocks, _NUM_Q)
    cb_spec = pl.BlockSpec((1, _K, _D), lambda i, q: (q, 0, 0))
    qout, idx, loss_part = pl.pallas_call(
        _rvq_block,
        grid=grid,
        in_specs=[
            pl.BlockSpec((_R, _D), lambda i, q: (i, 0)),
            cb_spec, cb_spec, cb_spec, cb_spec,
        ],
        out_specs=[
            pl.BlockSpec((_R, _D), lambda i, q: (i, 0)),
            pl.BlockSpec((1, 1, 1, _R), lambda i, q: (q, i, 0, 0)),
            pl.BlockSpec((1, 1, 1, 1), lambda i, q: (i, q, 0, 0)),
        ],
        out_shape=[
            jax.ShapeDtypeStruct((n, _D), jnp.float32),
            jax.ShapeDtypeStruct((_NUM_Q, nblocks, 1, _R), jnp.int32),
            jax.ShapeDtypeStruct((nblocks, _NUM_Q, 1, 1), jnp.float32),
        ],
        scratch_shapes=[pltpu.VMEM((_R, _D), jnp.float32)],
        compiler_params=pltpu.CompilerParams(
            dimension_semantics=("arbitrary", "arbitrary")),
    )(xf, codebooks, cb_hi16, cb_mid16, cb_lo16)

    quantized_out = qout.reshape(orig_shape)
    indices = idx.reshape(_NUM_Q, *orig_shape[:-1])
    losses = loss_part.reshape(nblocks, _NUM_Q).sum(axis=0) * (
        _COMMITMENT / float(n * _D))
    return quantized_out, indices, losses


# speedup vs baseline: 1.5662x; 1.0745x over previous
import functools

import jax
import jax.numpy as jnp
from jax.experimental import pallas as pl
from jax.experimental.pallas import tpu as pltpu

_NUM_Q = 8
_K = 8192
_D = 64
_COMMITMENT = 1.0
_R = 256


def _rvq_block(x_ref, cb_ref, cbhi_ref, cbmid_ref, cblo_ref,
               qout_ref, idx_ref, loss_ref, res_ref):
    q = pl.program_id(1)

    @pl.when(q == 0)
    def _init():
        res_ref[...] = x_ref[...]

    r = res_ref[...]
    cb = cb_ref[0]

    rn = jnp.sum(r * r, axis=1, keepdims=True)
    cbn = jnp.sum(cb * cb, axis=1)[None, :]
    mm = jax.lax.dot_general(
        r.astype(jnp.bfloat16), cb.astype(jnp.bfloat16),
        (((1,), (1,)), ((), ())),
        preferred_element_type=jnp.float32)
    neg_dist = -(rn - 2.0 * mm + cbn)

    idx = jnp.argmax(neg_dist, axis=1)
    iota = jax.lax.broadcasted_iota(jnp.int32, (_R, _K), 1)

    onehot = (iota == idx[:, None]).astype(jnp.bfloat16)
    dims = (((1,), (0,)), ((), ()))
    quant = (jax.lax.dot_general(onehot, cbhi_ref[0], dims,
                                 preferred_element_type=jnp.float32)
             + jax.lax.dot_general(onehot, cbmid_ref[0], dims,
                                   preferred_element_type=jnp.float32)
             ) + jax.lax.dot_general(onehot, cblo_ref[0], dims,
                                     preferred_element_type=jnp.float32)

    new_res = r - quant
    res_ref[...] = new_res

    idx_ref[...] = idx.reshape(1, 1, 1, _R)
    loss_ref[...] = jnp.sum(new_res * new_res).reshape(1, 1, 1, 1)

    @pl.when(q == 0)
    def _first():
        qout_ref[...] = quant

    @pl.when(q != 0)
    def _acc():
        qout_ref[...] = qout_ref[...] + quant


@functools.partial(jax.jit, static_argnames=())
def kernel(x, codebooks):
    orig_shape = x.shape
    n = orig_shape[0] * orig_shape[1]
    xf = x.reshape(n, _D)
    nblocks = n // _R

    cb_hi16 = codebooks.astype(jnp.bfloat16)
    cb_hi = cb_hi16.astype(jnp.float32)
    cb_mid16 = (codebooks - cb_hi).astype(jnp.bfloat16)
    cb_mid = cb_mid16.astype(jnp.float32)
    cb_lo16 = (codebooks - cb_hi - cb_mid).astype(jnp.bfloat16)

    grid = (nblocks, _NUM_Q)
    cb_spec = pl.BlockSpec((1, _K, _D), lambda i, q: (q, 0, 0))
    qout, idx, loss_part = pl.pallas_call(
        _rvq_block,
        grid=grid,
        in_specs=[
            pl.BlockSpec((_R, _D), lambda i, q: (i, 0)),
            cb_spec, cb_spec, cb_spec, cb_spec,
        ],
        out_specs=[
            pl.BlockSpec((_R, _D), lambda i, q: (i, 0)),
            pl.BlockSpec((1, 1, 1, _R), lambda i, q: (q, i, 0, 0)),
            pl.BlockSpec((1, 1, 1, 1), lambda i, q: (i, q, 0, 0)),
        ],
        out_shape=[
            jax.ShapeDtypeStruct((n, _D), jnp.float32),
            jax.ShapeDtypeStruct((_NUM_Q, nblocks, 1, _R), jnp.int32),
            jax.ShapeDtypeStruct((nblocks, _NUM_Q, 1, 1), jnp.float32),
        ],
        scratch_shapes=[pltpu.VMEM((_R, _D), jnp.float32)],
        compiler_params=pltpu.CompilerParams(
            dimension_semantics=("arbitrary", "arbitrary")),
    )(xf, codebooks, cb_hi16, cb_mid16, cb_lo16)

    quantized_out = qout.reshape(orig_shape)
    indices = idx.reshape(_NUM_Q, *orig_shape[:-1])
    losses = loss_part.reshape(nblocks, _NUM_Q).sum(axis=0) * (
        _COMMITMENT / float(n * _D))
    return quantized_out, indices, losses


# speedup vs baseline: 2.3055x; 1.4720x over previous
import functools

import jax
import jax.numpy as jnp
from jax.experimental import pallas as pl
from jax.experimental.pallas import tpu as pltpu

_NUM_Q = 8
_K = 8192
_D = 64
_COMMITMENT = 1.0
_R = 256


def _rvq_block(x_ref, cba_ref, cbhi_ref, cbmid_ref, cblo_ref,
               qout_ref, idx_ref, loss_ref, res_ref):
    q = pl.program_id(1)

    @pl.when(q == 0)
    def _init():
        res_ref[...] = x_ref[...]

    r = res_ref[...]
    cba = cba_ref[0]

    lane = jax.lax.broadcasted_iota(jnp.int32, (_R, _D), 1)
    ones2 = jnp.where(lane < 2, 1.0, 0.0).astype(jnp.bfloat16)
    r_aug = jnp.concatenate([r.astype(jnp.bfloat16), ones2], axis=1)

    scores = jax.lax.dot_general(
        r_aug, cba, (((1,), (1,)), ((), ())),
        preferred_element_type=jnp.float32)

    idx = jnp.argmax(scores, axis=1)
    iota = jax.lax.broadcasted_iota(jnp.int32, (_R, _K), 1)
    onehot = (iota == idx[:, None]).astype(jnp.bfloat16)
    dims = (((1,), (0,)), ((), ()))
    quant = (jax.lax.dot_general(onehot, cbhi_ref[0], dims,
                                 preferred_element_type=jnp.float32)
             + jax.lax.dot_general(onehot, cbmid_ref[0], dims,
                                   preferred_element_type=jnp.float32)
             ) + jax.lax.dot_general(onehot, cblo_ref[0], dims,
                                     preferred_element_type=jnp.float32)

    new_res = r - quant
    res_ref[...] = new_res

    idx_ref[...] = idx.reshape(1, 1, 1, _R)
    loss_ref[...] = jnp.sum(new_res * new_res).reshape(1, 1, 1, 1)

    @pl.when(q == 0)
    def _first():
        qout_ref[...] = quant

    @pl.when(q != 0)
    def _acc():
        qout_ref[...] = qout_ref[...] + quant


@functools.partial(jax.jit, static_argnames=())
def kernel(x, codebooks):
    orig_shape = x.shape
    n = orig_shape[0] * orig_shape[1]
    xf = x.reshape(n, _D)
    nblocks = n // _R

    cb_hi16 = codebooks.astype(jnp.bfloat16)
    cb_hi = cb_hi16.astype(jnp.float32)
    cb_mid16 = (codebooks - cb_hi).astype(jnp.bfloat16)
    cb_mid = cb_mid16.astype(jnp.float32)
    cb_lo16 = (codebooks - cb_hi - cb_mid).astype(jnp.bfloat16)

    bias = -0.5 * jnp.sum(codebooks * codebooks, axis=2)
    b_hi16 = bias.astype(jnp.bfloat16)
    b_lo16 = (bias - b_hi16.astype(jnp.float32)).astype(jnp.bfloat16)
    zeros = jnp.zeros((_NUM_Q, _K, _D - 2), jnp.bfloat16)
    cb_aug = jnp.concatenate(
        [cb_hi16, b_hi16[..., None], b_lo16[..., None], zeros], axis=2)

    grid = (nblocks, _NUM_Q)
    cb_spec = pl.BlockSpec((1, _K, _D), lambda i, q: (q, 0, 0))
    qout, idx, loss_part = pl.pallas_call(
        _rvq_block,
        grid=grid,
        in_specs=[
            pl.BlockSpec((_R, _D), lambda i, q: (i, 0)),
            pl.BlockSpec((1, _K, 2 * _D), lambda i, q: (q, 0, 0)),
            cb_spec, cb_spec, cb_spec,
        ],
        out_specs=[
            pl.BlockSpec((_R, _D), lambda i, q: (i, 0)),
            pl.BlockSpec((1, 1, 1, _R), lambda i, q: (q, i, 0, 0)),
            pl.BlockSpec((1, 1, 1, 1), lambda i, q: (i, q, 0, 0)),
        ],
        out_shape=[
            jax.ShapeDtypeStruct((n, _D), jnp.float32),
            jax.ShapeDtypeStruct((_NUM_Q, nblocks, 1, _R), jnp.int32),
            jax.ShapeDtypeStruct((nblocks, _NUM_Q, 1, 1), jnp.float32),
        ],
        scratch_shapes=[pltpu.VMEM((_R, _D), jnp.float32)],
        compiler_params=pltpu.CompilerParams(
            dimension_semantics=("arbitrary", "arbitrary")),
    )(xf, cb_aug, cb_hi16, cb_mid16, cb_lo16)

    quantized_out = qout.reshape(orig_shape)
    indices = idx.reshape(_NUM_Q, *orig_shape[:-1])
    losses = loss_part.reshape(nblocks, _NUM_Q).sum(axis=0) * (
        _COMMITMENT / float(n * _D))
    return quantized_out, indices, losses
